# baseline (speedup 1.0000x reference)
import functools

import jax
import jax.numpy as jnp
from jax import lax
from jax.experimental import pallas as pl
from jax.experimental.pallas import tpu as pltpu
from jax.experimental.pallas import tpu_sc as plsc

VOCAB = 100000
HIDDEN = 128
MAX_POS = 2048
B = 4
L = 2048
EPS = 1e-5

N = B * L
NW = 32
TPW = N // NW
ICH = 128
NCH = TPW // ICH
HREG = HIDDEN // 16
LANES = 16
GRP = 16
NGRP = TPW // GRP
GPC = ICH // GRP
ACC = TPW + LANES


def _rsqrt(xv):
    i = plsc.bitcast(xv, jnp.int32)
    i = jnp.int32(0x5F3759DF) - (i >> 1)
    y = plsc.bitcast(i, jnp.float32)
    half = xv * jnp.float32(0.5)
    for _ in range(3):
        y = y * (jnp.float32(1.5) - half * y * y)
    return y


def _make_kernel():
    mesh = plsc.VectorSubcoreMesh(core_axis_name="c", subcore_axis_name="s")

    @functools.partial(
        pl.kernel,
        mesh=mesh,
        out_type=jax.ShapeDtypeStruct((N, HIDDEN), jnp.float32),
        compiler_params=pltpu.CompilerParams(needs_layout_passes=False,
                                             skip_device_barrier=True),
        scratch_types=[
            pltpu.VMEM((NCH, ICH), jnp.int32),
            pltpu.VMEM((NCH, ICH), jnp.int32),
            pltpu.VMEM((TPW, HIDDEN), jnp.float32),
            pltpu.VMEM((TPW, HIDDEN), jnp.float32),
            pltpu.VMEM((TPW, HIDDEN), jnp.float32),
            pltpu.VMEM((2, HIDDEN), jnp.float32),
            pltpu.VMEM((TPW + 8,), jnp.float32),
            pltpu.VMEM((HIDDEN,), jnp.float32),
            pltpu.VMEM((HIDDEN,), jnp.float32),
            pltpu.VMEM((ACC,), jnp.float32),
            pltpu.VMEM((ACC,), jnp.float32),
            pltpu.VMEM((LANES + 8,), jnp.float32),
            pltpu.VMEM((LANES + 8,), jnp.float32),
            pltpu.SemaphoreType.DMA,
            pltpu.SemaphoreType.DMA,
            pltpu.SemaphoreType.DMA,
            pltpu.SemaphoreType.DMA,
            pltpu.SemaphoreType.DMA,
        ],
    )
    def emb_kernel(text_hbm, seg_hbm, word_hbm, pos_hbm, segtab_hbm,
                   gamma_hbm, beta_hbm, out_hbm,
                   idx_v, segidx_v, words_v, ebuf_v, pos_v, segtab_v, segf_v,
                   gamma_v, beta_v, ssum_v, ssq_v, rstd_v, m2_v,
                   sem_i, sem_b, sem_w0, sem_w1, sem_o):
        wid = lax.axis_index("s") * 2 + lax.axis_index("c")
        base = wid * TPW

        ci1 = pltpu.async_copy(text_hbm.at[wid], idx_v, sem_i)
        ci2 = pltpu.async_copy(seg_hbm.at[wid], segidx_v, sem_i)
        pos_base = lax.rem(base, L)
        cb = [pltpu.async_copy(pos_hbm.at[pl.ds(pos_base, TPW)], pos_v,
                               sem_b),
              pltpu.async_copy(gamma_hbm, gamma_v, sem_b),
              pltpu.async_copy(beta_hbm, beta_v, sem_b),
              pltpu.async_copy(segtab_hbm, segtab_v, sem_b)]

        lane = lax.iota(jnp.int32, LANES)
        zero16 = jnp.zeros((LANES,), dtype=jnp.float32)
        for i in range(1, ACC // LANES):
            plsc.store_scatter(ssum_v, [lane + (i * LANES)], zero16)
            plsc.store_scatter(ssq_v, [lane + (i * LANES)], zero16)

        ci1.wait()
        ci2.wait()
        cw = [pltpu.async_copy(word_hbm.at[idx_v.at[0]],
                               words_v.at[pl.ds(0, ICH)], sem_w0),
              pltpu.async_copy(word_hbm.at[idx_v.at[1]],
                               words_v.at[pl.ds(ICH, ICH)], sem_w1)]

        for c in cb:
            c.wait()
        for j in range(NCH):
            for k in range(ICH // LANES):
                iv = segidx_v[j, pl.ds(k * LANES, LANES)]
                segf_v[pl.ds(1 + j * ICH + k * LANES, LANES)] = \
                    iv.astype(jnp.float32)

        seg0 = [segtab_v[0, pl.ds(h * LANES, LANES)] for h in range(HREG)]
        segd = [segtab_v[1, pl.ds(h * LANES, LANES)] - seg0[h]
                for h in range(HREG)]
        inv_h = jnp.float32(1.0 / HIDDEN)

        def pass_a(g, carry):
            t0 = g * GRP
            for tt in range(GRP):
                t = t0 + tt
                sf = plsc.load_gather(
                    segf_v, [jnp.broadcast_to(t + 1, (LANES,))])
                e = []
                for h in range(HREG):
                    hs = pl.ds(h * LANES, LANES)
                    v = (words_v[t, hs] + pos_v[t, hs]
                         + (seg0[h] + sf * segd[h]))
                    e.append(v)
                tot = e[0]
                sq = e[0] * e[0]
                for h in range(1, HREG):
                    tot = tot + e[h]
                    sq = sq + e[h] * e[h]
                slot = jnp.broadcast_to(t + LANES, (LANES,))
                plsc.addupdate_scatter(ssum_v, [slot], tot)
                plsc.addupdate_scatter(ssq_v, [slot], sq)
                for h in range(HREG):
                    ebuf_v[t, pl.ds(h * LANES, LANES)] = e[h]
            return carry

        gammas = [gamma_v[pl.ds(h * LANES, LANES)] for h in range(HREG)]
        betas = [beta_v[pl.ds(h * LANES, LANES)] for h in range(HREG)]
        splats = [jnp.full((LANES,), tt + 1, dtype=jnp.int32)
                  for tt in range(GRP)]

        def pass_b(g, carry):
            t0 = g * GRP
            s = ssum_v[pl.ds(t0 + LANES, LANES)]
            ss = ssq_v[pl.ds(t0 + LANES, LANES)]
            mean = s * inv_h
            var = ss * inv_h - mean * mean
            rstd = _rsqrt(var + jnp.float32(EPS))
            rstd_v[pl.ds(1, LANES)] = rstd
            m2_v[pl.ds(1, LANES)] = mean * rstd
            for tt in range(GRP):
                t = t0 + tt
                r = plsc.load_gather(rstd_v, [splats[tt]])
                m = plsc.load_gather(m2_v, [splats[tt]])
                for h in range(HREG):
                    hs = pl.ds(h * LANES, LANES)
                    words_v[t, hs] = (ebuf_v[t, hs] * r - m) * gammas[h] \
                        + betas[h]
            return carry

        cw[0].wait()
        lax.fori_loop(0, GPC, pass_a, 0)
        lax.fori_loop(0, GPC, pass_b, 0)
        co = pltpu.async_copy(words_v.at[pl.ds(0, ICH)],
                              out_hbm.at[pl.ds(base, ICH)], sem_o)
        cw[1].wait()
        lax.fori_loop(GPC, NGRP, pass_a, 0)
        lax.fori_loop(GPC, NGRP, pass_b, 0)
        pltpu.sync_copy(words_v.at[pl.ds(ICH, ICH)],
                        out_hbm.at[pl.ds(base + ICH, ICH)])
        co.wait()

    return emb_kernel


_emb_kernel = _make_kernel()


def kernel(batch_text_idx, batch_seg_idx, word_table, pos_table, seg_table,
           ln_gamma, ln_beta):
    text = batch_text_idx.reshape(NW, NCH, ICH).astype(jnp.int32)
    seg = batch_seg_idx.reshape(NW, NCH, ICH).astype(jnp.int32)
    out = _emb_kernel(text, seg, word_table, pos_table, seg_table,
                      ln_gamma, ln_beta)
    return out.reshape(B, L, HIDDEN)


# speedup vs baseline: 1.0472x; 1.0472x over previous
import functools

import jax
import jax.numpy as jnp
from jax import lax
from jax.experimental import pallas as pl
from jax.experimental.pallas import tpu as pltpu
from jax.experimental.pallas import tpu_sc as plsc

VOCAB = 100000
HIDDEN = 128
MAX_POS = 2048
B = 4
L = 2048
EPS = 1e-5

N = B * L
NW = 32
TPW = N // NW
ICH = 128
NCH = TPW // ICH
HREG = HIDDEN // 16
LANES = 16
GRP = 16
NGRP = TPW // GRP
GPC = ICH // GRP
ACC = TPW + LANES


def _rsqrt(xv):
    i = plsc.bitcast(xv, jnp.int32)
    i = jnp.int32(0x5F3759DF) - (i >> 1)
    y = plsc.bitcast(i, jnp.float32)
    half = xv * jnp.float32(0.5)
    for _ in range(3):
        y = y * (jnp.float32(1.5) - half * y * y)
    return y


def _make_kernel():
    mesh = plsc.VectorSubcoreMesh(core_axis_name="c", subcore_axis_name="s")

    @functools.partial(
        pl.kernel,
        mesh=mesh,
        out_type=jax.ShapeDtypeStruct((N, HIDDEN), jnp.float32),
        compiler_params=pltpu.CompilerParams(needs_layout_passes=False,
                                             skip_device_barrier=True),
        scratch_types=[
            pltpu.VMEM((NCH, ICH), jnp.int32),
            pltpu.VMEM((TPW,), jnp.int32),
            pltpu.VMEM((TPW, HIDDEN), jnp.float32),
            pltpu.VMEM((TPW, HIDDEN), jnp.float32),
            pltpu.VMEM((TPW, HIDDEN), jnp.float32),
            pltpu.VMEM((2, HIDDEN), jnp.float32),
            pltpu.VMEM((TPW + 8,), jnp.float32),
            pltpu.VMEM((HIDDEN,), jnp.float32),
            pltpu.VMEM((HIDDEN,), jnp.float32),
            pltpu.VMEM((ACC,), jnp.float32),
            pltpu.VMEM((ACC,), jnp.float32),
            pltpu.VMEM((LANES + 8,), jnp.float32),
            pltpu.VMEM((LANES + 8,), jnp.float32),
            pltpu.SemaphoreType.DMA,
            pltpu.SemaphoreType.DMA,
            pltpu.SemaphoreType.DMA,
            pltpu.SemaphoreType.DMA,
            pltpu.SemaphoreType.DMA,
        ],
    )
    def emb_kernel(text_hbm, seg_hbm, word_hbm, pos_hbm, segtab_hbm,
                   gamma_hbm, beta_hbm, out_hbm,
                   idx_v, segidx_v, words_v, ebuf_v, pos_v, segtab_v, segf_v,
                   gamma_v, beta_v, ssum_v, ssq_v, rstd_v, m2_v,
                   sem_i, sem_b, sem_w0, sem_w1, sem_o):
        wid = lax.axis_index("s") * 2 + lax.axis_index("c")
        base = wid * TPW
        bb = wid // (L // TPW)
        col = lax.rem(base, L)

        cis = [pltpu.async_copy(text_hbm.at[bb, pl.ds(col + j * ICH, ICH)],
                                idx_v.at[j], sem_i)
               for j in range(NCH)]
        cis.append(pltpu.async_copy(seg_hbm.at[bb, pl.ds(col, TPW)],
                                    segidx_v, sem_i))
        pos_base = col
        cb = [pltpu.async_copy(pos_hbm.at[pl.ds(pos_base, TPW)], pos_v,
                               sem_b),
              pltpu.async_copy(gamma_hbm, gamma_v, sem_b),
              pltpu.async_copy(beta_hbm, beta_v, sem_b),
              pltpu.async_copy(segtab_hbm, segtab_v, sem_b)]

        lane = lax.iota(jnp.int32, LANES)
        zero16 = jnp.zeros((LANES,), dtype=jnp.float32)
        for i in range(1, ACC // LANES):
            plsc.store_scatter(ssum_v, [lane + (i * LANES)], zero16)
            plsc.store_scatter(ssq_v, [lane + (i * LANES)], zero16)

        for c in cis:
            c.wait()
        cw = [pltpu.async_copy(word_hbm.at[idx_v.at[0]],
                               words_v.at[pl.ds(0, ICH)], sem_w0),
              pltpu.async_copy(word_hbm.at[idx_v.at[1]],
                               words_v.at[pl.ds(ICH, ICH)], sem_w1)]

        for c in cb:
            c.wait()
        for k in range(TPW // LANES):
            iv = segidx_v[pl.ds(k * LANES, LANES)]
            segf_v[pl.ds(1 + k * LANES, LANES)] = iv.astype(jnp.float32)

        seg0 = [segtab_v[0, pl.ds(h * LANES, LANES)] for h in range(HREG)]
        segd = [segtab_v[1, pl.ds(h * LANES, LANES)] - seg0[h]
                for h in range(HREG)]
        inv_h = jnp.float32(1.0 / HIDDEN)

        def pass_a(g, carry):
            t0 = g * GRP
            for tt in range(GRP):
                t = t0 + tt
                sf = plsc.load_gather(
                    segf_v, [jnp.broadcast_to(t + 1, (LANES,))])
                e = []
                for h in range(HREG):
                    hs = pl.ds(h * LANES, LANES)
                    v = (words_v[t, hs] + pos_v[t, hs]
                         + (seg0[h] + sf * segd[h]))
                    e.append(v)
                tot = e[0]
                sq = e[0] * e[0]
                for h in range(1, HREG):
                    tot = tot + e[h]
                    sq = sq + e[h] * e[h]
                slot = jnp.broadcast_to(t + LANES, (LANES,))
                plsc.addupdate_scatter(ssum_v, [slot], tot)
                plsc.addupdate_scatter(ssq_v, [slot], sq)
                for h in range(HREG):
                    ebuf_v[t, pl.ds(h * LANES, LANES)] = e[h]
            return carry

        gammas = [gamma_v[pl.ds(h * LANES, LANES)] for h in range(HREG)]
        betas = [beta_v[pl.ds(h * LANES, LANES)] for h in range(HREG)]
        splats = [jnp.full((LANES,), tt + 1, dtype=jnp.int32)
                  for tt in range(GRP)]

        def pass_b(g, carry):
            t0 = g * GRP
            s = ssum_v[pl.ds(t0 + LANES, LANES)]
            ss = ssq_v[pl.ds(t0 + LANES, LANES)]
            mean = s * inv_h
            var = ss * inv_h - mean * mean
            rstd = _rsqrt(var + jnp.float32(EPS))
            rstd_v[pl.ds(1, LANES)] = rstd
            m2_v[pl.ds(1, LANES)] = mean * rstd
            for tt in range(GRP):
                t = t0 + tt
                r = plsc.load_gather(rstd_v, [splats[tt]])
                m = plsc.load_gather(m2_v, [splats[tt]])
                for h in range(HREG):
                    hs = pl.ds(h * LANES, LANES)
                    words_v[t, hs] = (ebuf_v[t, hs] * r - m) * gammas[h] \
                        + betas[h]
            return carry

        cw[0].wait()
        cw[1].wait()
        lax.fori_loop(0, NGRP, pass_a, 0)
        lax.fori_loop(0, NGRP, pass_b, 0)
        pltpu.sync_copy(words_v, out_hbm.at[pl.ds(base, TPW)])

    return emb_kernel


_emb_kernel = _make_kernel()


def kernel(batch_text_idx, batch_seg_idx, word_table, pos_table, seg_table,
           ln_gamma, ln_beta):
    out = _emb_kernel(batch_text_idx.astype(jnp.int32),
                      batch_seg_idx.astype(jnp.int32),
                      word_table, pos_table, seg_table, ln_gamma, ln_beta)
    return out.reshape(B, L, HIDDEN)


# speedup vs baseline: 1.1768x; 1.1237x over previous
import functools

import jax
import jax.numpy as jnp
from jax import lax
from jax.experimental import pallas as pl
from jax.experimental.pallas import tpu as pltpu
from jax.experimental.pallas import tpu_sc as plsc

VOCAB = 100000
HIDDEN = 128
MAX_POS = 2048
B = 4
L = 2048
EPS = 1e-5

N = B * L
NW = 32
TPW = N // NW
ICH = 128
NCH = TPW // ICH
HREG = HIDDEN // 16
LANES = 16
GRP = 16
NGRP = TPW // GRP
GPC = ICH // GRP
ACC = TPW + LANES


def _rsqrt(xv):
    i = plsc.bitcast(xv, jnp.int32)
    i = jnp.int32(0x5F3759DF) - (i >> 1)
    y = plsc.bitcast(i, jnp.float32)
    half = xv * jnp.float32(0.5)
    for _ in range(3):
        y = y * (jnp.float32(1.5) - half * y * y)
    return y


def _make_kernel():
    mesh = plsc.VectorSubcoreMesh(core_axis_name="c", subcore_axis_name="s")

    @functools.partial(
        pl.kernel,
        mesh=mesh,
        out_type=jax.ShapeDtypeStruct((N, HIDDEN), jnp.float32),
        compiler_params=pltpu.CompilerParams(needs_layout_passes=False,
                                             skip_device_barrier=True),
        scratch_types=[
            pltpu.VMEM((NCH, ICH), jnp.int32),
            pltpu.VMEM((TPW,), jnp.int32),
            pltpu.VMEM((TPW, HIDDEN), jnp.float32),
            pltpu.VMEM((TPW, HIDDEN), jnp.float32),
            pltpu.VMEM((TPW, HIDDEN), jnp.float32),
            pltpu.VMEM((2, HIDDEN), jnp.float32),
            pltpu.VMEM((TPW + 8,), jnp.float32),
            pltpu.VMEM((HIDDEN,), jnp.float32),
            pltpu.VMEM((HIDDEN,), jnp.float32),
            pltpu.VMEM((ACC,), jnp.float32),
            pltpu.VMEM((ACC,), jnp.float32),
            pltpu.VMEM((ACC,), jnp.float32),
            pltpu.VMEM((ACC,), jnp.float32),
            pltpu.SemaphoreType.DMA,
            pltpu.SemaphoreType.DMA,
            pltpu.SemaphoreType.DMA,
            pltpu.SemaphoreType.DMA,
            pltpu.SemaphoreType.DMA,
        ],
    )
    def emb_kernel(text_hbm, seg_hbm, word_hbm, pos_hbm, segtab_hbm,
                   gamma_hbm, beta_hbm, out_hbm,
                   idx_v, segidx_v, words_v, ebuf_v, pos_v, segtab_v, segf_v,
                   gamma_v, beta_v, ssum_v, ssq_v, rstd_v, m2_v,
                   sem_i, sem_b, sem_w0, sem_w1, sem_o):
        wid = lax.axis_index("s") * 2 + lax.axis_index("c")
        base = wid * TPW
        bb = wid // (L // TPW)
        col = lax.rem(base, L)

        cis = [pltpu.async_copy(text_hbm.at[bb, pl.ds(col + j * ICH, ICH)],
                                idx_v.at[j], sem_i)
               for j in range(NCH)]
        cis.append(pltpu.async_copy(seg_hbm.at[bb, pl.ds(col, TPW)],
                                    segidx_v, sem_i))
        pos_base = col
        cb = [pltpu.async_copy(pos_hbm.at[pl.ds(pos_base, TPW)], pos_v,
                               sem_b),
              pltpu.async_copy(gamma_hbm, gamma_v, sem_b),
              pltpu.async_copy(beta_hbm, beta_v, sem_b),
              pltpu.async_copy(segtab_hbm, segtab_v, sem_b)]

        lane = lax.iota(jnp.int32, LANES)
        zero16 = jnp.zeros((LANES,), dtype=jnp.float32)
        for i in range(1, ACC // LANES):
            plsc.store_scatter(ssum_v, [lane + (i * LANES)], zero16)
            plsc.store_scatter(ssq_v, [lane + (i * LANES)], zero16)

        for c in cis:
            c.wait()
        cw = [pltpu.async_copy(word_hbm.at[idx_v.at[0]],
                               words_v.at[pl.ds(0, ICH)], sem_w0),
              pltpu.async_copy(word_hbm.at[idx_v.at[1]],
                               words_v.at[pl.ds(ICH, ICH)], sem_w1)]

        for c in cb:
            c.wait()
        for k in range(TPW // LANES):
            iv = segidx_v[pl.ds(k * LANES, LANES)]
            segf_v[pl.ds(1 + k * LANES, LANES)] = iv.astype(jnp.float32)

        seg0 = [segtab_v[0, pl.ds(h * LANES, LANES)] for h in range(HREG)]
        segd = [segtab_v[1, pl.ds(h * LANES, LANES)] - seg0[h]
                for h in range(HREG)]
        inv_h = jnp.float32(1.0 / HIDDEN)

        cw[0].wait()
        cw[1].wait()

        @plsc.parallel_loop(0, TPW, unroll=4)
        def pass_a(t):
            sf = plsc.load_gather(
                segf_v, [jnp.broadcast_to(t + 1, (LANES,))])
            e = []
            for h in range(HREG):
                hs = pl.ds(h * LANES, LANES)
                v = (words_v[t, hs] + pos_v[t, hs]
                     + (seg0[h] + sf * segd[h]))
                e.append(v)
            tot = e[0]
            sq = e[0] * e[0]
            for h in range(1, HREG):
                tot = tot + e[h]
                sq = sq + e[h] * e[h]
            slot = jnp.broadcast_to(t + LANES, (LANES,))
            plsc.addupdate_scatter(ssum_v, [slot], tot)
            plsc.addupdate_scatter(ssq_v, [slot], sq)
            for h in range(HREG):
                ebuf_v[t, pl.ds(h * LANES, LANES)] = e[h]

        gammas = [gamma_v[pl.ds(h * LANES, LANES)] for h in range(HREG)]
        betas = [beta_v[pl.ds(h * LANES, LANES)] for h in range(HREG)]

        @plsc.parallel_loop(0, NGRP, unroll=2)
        def pass_stats(g):
            t0 = g * GRP
            s = ssum_v[pl.ds(t0 + LANES, LANES)]
            ss = ssq_v[pl.ds(t0 + LANES, LANES)]
            mean = s * inv_h
            var = ss * inv_h - mean * mean
            rstd = _rsqrt(var + jnp.float32(EPS))
            rstd_v[pl.ds(t0 + LANES, LANES)] = rstd
            m2_v[pl.ds(t0 + LANES, LANES)] = mean * rstd

        @plsc.parallel_loop(0, TPW, unroll=4)
        def pass_b(t):
            slot = jnp.broadcast_to(t + LANES, (LANES,))
            r = plsc.load_gather(rstd_v, [slot])
            m = plsc.load_gather(m2_v, [slot])
            for h in range(HREG):
                hs = pl.ds(h * LANES, LANES)
                words_v[t, hs] = (ebuf_v[t, hs] * r - m) * gammas[h] \
                    + betas[h]

        pltpu.sync_copy(words_v, out_hbm.at[pl.ds(base, TPW)])

    return emb_kernel


_emb_kernel = _make_kernel()


def kernel(batch_text_idx, batch_seg_idx, word_table, pos_table, seg_table,
           ln_gamma, ln_beta):
    out = _emb_kernel(batch_text_idx.astype(jnp.int32),
                      batch_seg_idx.astype(jnp.int32),
                      word_table, pos_table, seg_table, ln_gamma, ln_beta)
    return out.reshape(B, L, HIDDEN)
